# baseline (speedup 1.0000x reference)
import jax
import jax.numpy as jnp
from jax import lax
from jax.experimental import pallas as pl
from jax.experimental.pallas import tpu as pltpu
from jax.experimental.pallas import tpu_sc as plsc

B = 4096
SEQ = 50
D = 128
NL = 16
NW = 32
ROWS_PER_W = B // NW
ROWS_PER_CHUNK = 2
IDX_PER_CHUNK = ROWS_PER_CHUNK * SEQ
CHUNKS = ROWS_PER_W // ROWS_PER_CHUNK
VCHUNKS = CHUNKS // 2
SROWS = VCHUNKS * ROWS_PER_CHUNK


def _sc_pool(idx_hbm, pat_hbm, zeros_hbm, table_hbm, out_hbm,
             idx_v, sidx_v, rows_v, pooled_v, zread_v, spmem,
             gv0, gv1, gs0, gs1, ssem, osem):
    c = lax.axis_index("c")
    s = lax.axis_index("s")
    wid = s * 2 + c
    gv = (gv0, gv1)
    gs = (gs0, gs1)
    pltpu.sync_copy(idx_hbm.at[wid], idx_v)
    pltpu.sync_copy(pat_hbm.at[s], sidx_v)
    pltpu.sync_copy(zeros_hbm, spmem.at[pl.ds(s * SROWS, SROWS)])

    for p in range(2):
        pltpu.async_copy(table_hbm.at[idx_v.at[p]], rows_v.at[p], gv[p])
        pltpu.async_copy(
            table_hbm.at[idx_v.at[VCHUNKS + p]], rows_v.at[2 + p], gs[p])

    def step_body(g2, carry):
        for p in range(2):
            g = g2 * 2 + p
            pltpu.make_async_copy(
                table_hbm.at[idx_v.at[0]], rows_v.at[2 + p], gs[p]).wait()
            pltpu.async_copy(
                rows_v.at[2 + p], spmem.at[sidx_v.at[g]], ssem, add=True)
            @pl.when(g >= 2)
            def _():
                pltpu.make_async_copy(
                    pooled_v.at[0],
                    out_hbm.at[pl.ds(0, ROWS_PER_CHUNK)], osem).wait()
            pltpu.make_async_copy(
                table_hbm.at[idx_v.at[0]], rows_v.at[p], gv[p]).wait()
            for r in range(ROWS_PER_CHUNK):
                def seq_body(t, accs, r=r, p=p):
                    base = r * SEQ + t * 10
                    out = list(accs)
                    for tt in range(10):
                        for l in range(D // NL):
                            out[l] = out[l] + rows_v[p, base + tt,
                                                     pl.ds(l * NL, NL)]
                    return tuple(out)

                accs = lax.fori_loop(
                    0, SEQ // 10, seq_body,
                    tuple(jnp.zeros((NL,), jnp.float32)
                          for _ in range(D // NL)))
                for l in range(D // NL):
                    pooled_v[p, r, pl.ds(l * NL, NL)] = (
                        accs[l] * (1.0 / SEQ))
            pltpu.make_async_copy(
                rows_v.at[2 + p], spmem.at[sidx_v.at[0]], ssem).wait()
            @pl.when(g + 2 < VCHUNKS)
            def _():
                pltpu.async_copy(
                    table_hbm.at[idx_v.at[g + 2]], rows_v.at[p], gv[p])
                pltpu.async_copy(
                    table_hbm.at[idx_v.at[VCHUNKS + g + 2]],
                    rows_v.at[2 + p], gs[p])
            pltpu.async_copy(
                pooled_v.at[p],
                out_hbm.at[pl.ds(wid * ROWS_PER_W + g * ROWS_PER_CHUNK,
                                 ROWS_PER_CHUNK)], osem)
        return carry

    lax.fori_loop(0, VCHUNKS // 2, step_body, 0)
    for _ in range(2):
        pltpu.make_async_copy(
            pooled_v.at[0], out_hbm.at[pl.ds(0, ROWS_PER_CHUNK)],
            osem).wait()
    pltpu.sync_copy(spmem.at[pl.ds(s * SROWS, SROWS)], zread_v)

    def scale_body(i, carry):
        for l in range(D // NL):
            zread_v[i, pl.ds(l * NL, NL)] = (
                zread_v[i, pl.ds(l * NL, NL)] * (1.0 / SEQ))
        return carry

    lax.fori_loop(0, SROWS, scale_body, 0)
    pltpu.sync_copy(
        zread_v, out_hbm.at[pl.ds(wid * ROWS_PER_W + SROWS, SROWS)])


def _mlp_body(x_ref, w1_ref, b1_ref, w2_ref, b2_ref, w3_ref, b3_ref, o_ref):
    x = x_ref[...]
    h = jnp.maximum(
        jnp.dot(x, w1_ref[...], preferred_element_type=jnp.float32)
        + b1_ref[...], 0.0)
    h = jnp.maximum(
        jnp.dot(h, w2_ref[...], preferred_element_type=jnp.float32)
        + b2_ref[...], 0.0)
    o_ref[...] = (
        jnp.dot(h, w3_ref[...], preferred_element_type=jnp.float32)
        + b3_ref[...])


def kernel(node_sequence, table, W1, b1, W2, b2, W3, b3):
    idx = node_sequence.astype(jnp.int32).reshape(NW, CHUNKS, IDX_PER_CHUNK)
    pat = (jnp.arange(16, dtype=jnp.int32)[:, None, None] * SROWS
           + jnp.arange(VCHUNKS, dtype=jnp.int32)[None, :, None]
           * ROWS_PER_CHUNK
           + (jnp.arange(IDX_PER_CHUNK, dtype=jnp.int32)[None, None, :]
              >= SEQ).astype(jnp.int32))
    zeros = jnp.zeros((SROWS, D), jnp.float32)

    mesh = plsc.VectorSubcoreMesh(core_axis_name="c", subcore_axis_name="s")
    pooled = pl.kernel(
        _sc_pool,
        mesh=mesh,
        out_type=jax.ShapeDtypeStruct((B, D), jnp.float32),
        scratch_types=[
            pltpu.VMEM((CHUNKS, IDX_PER_CHUNK), jnp.int32),
            pltpu.VMEM((VCHUNKS, IDX_PER_CHUNK), jnp.int32),
            pltpu.VMEM((4, IDX_PER_CHUNK, D), jnp.float32),
            pltpu.VMEM((2, ROWS_PER_CHUNK, D), jnp.float32),
            pltpu.VMEM((SROWS, D), jnp.float32),
            pltpu.VMEM_SHARED((16 * SROWS, D), jnp.float32),
            pltpu.SemaphoreType.DMA,
            pltpu.SemaphoreType.DMA,
            pltpu.SemaphoreType.DMA,
            pltpu.SemaphoreType.DMA,
            pltpu.SemaphoreType.DMA,
            pltpu.SemaphoreType.DMA,
        ],
    )(idx, pat, zeros, table)

    bt = 512
    out = pl.pallas_call(
        _mlp_body,
        grid=(B // bt,),
        in_specs=[
            pl.BlockSpec((bt, D), lambda i: (i, 0)),
            pl.BlockSpec((D, 128), lambda i: (0, 0)),
            pl.BlockSpec((1, 128), lambda i: (0, 0)),
            pl.BlockSpec((128, 64), lambda i: (0, 0)),
            pl.BlockSpec((1, 64), lambda i: (0, 0)),
            pl.BlockSpec((64, 1), lambda i: (0, 0)),
            pl.BlockSpec((1, 1), lambda i: (0, 0)),
        ],
        out_specs=pl.BlockSpec((bt, 1), lambda i: (i, 0)),
        out_shape=jax.ShapeDtypeStruct((B, 1), jnp.float32),
    )(pooled, W1, b1.reshape(1, 128), W2, b2.reshape(1, 64),
      W3, b3.reshape(1, 1))
    return out


# speedup vs baseline: 1.0593x; 1.0593x over previous
import functools

import jax
import jax.numpy as jnp
from jax import lax
from jax.experimental import pallas as pl
from jax.experimental.pallas import tpu as pltpu
from jax.experimental.pallas import tpu_sc as plsc

B = 4096
SEQ = 50
D = 128
NL = 16
NW = 32
ROWS_PER_W = B // NW
ROWS_PER_CHUNK = 2
IDX_PER_CHUNK = ROWS_PER_CHUNK * SEQ
CHUNKS = ROWS_PER_W // ROWS_PER_CHUNK


NBUF = 2


def _sc_pool(idx_hbm, table_hbm, out_hbm, idx_v, rows_v, pooled_v,
             sem0, sem1, osem):
    c = lax.axis_index("c")
    s = lax.axis_index("s")
    wid = s * 2 + c
    sems = (sem0, sem1)
    pltpu.sync_copy(idx_hbm.at[wid], idx_v)

    for b in range(NBUF):
        pltpu.async_copy(table_hbm.at[idx_v.at[b]], rows_v.at[b], sems[b])

    def step_body(g, carry):
        @pl.when(g >= 2)
        def _():
            pltpu.make_async_copy(
                pooled_v.at[0], out_hbm.at[pl.ds(0, NBUF * ROWS_PER_CHUNK)],
                osem).wait()
        for b in range(NBUF):
            j = g * NBUF + b
            pltpu.make_async_copy(
                table_hbm.at[idx_v.at[0]], rows_v.at[b], sems[b]).wait()
            for r in range(ROWS_PER_CHUNK):
                def seq_body(t, accs, r=r, b=b):
                    base = r * SEQ + t * 10
                    out = list(accs)
                    for tt in range(10):
                        for l in range(D // NL):
                            out[l] = out[l] + rows_v[b, base + tt,
                                                     pl.ds(l * NL, NL)]
                    return tuple(out)

                accs = lax.fori_loop(
                    0, SEQ // 10, seq_body,
                    tuple(jnp.zeros((NL,), jnp.float32)
                          for _ in range(D // NL)))
                for l in range(D // NL):
                    pooled_v[g % 2, b * ROWS_PER_CHUNK + r,
                             pl.ds(l * NL, NL)] = (accs[l] * (1.0 / SEQ))
            @pl.when(j + NBUF < CHUNKS)
            def _():
                pltpu.async_copy(
                    table_hbm.at[idx_v.at[j + NBUF]], rows_v.at[b], sems[b])
        out_base = wid * ROWS_PER_W + g * (NBUF * ROWS_PER_CHUNK)
        pltpu.async_copy(
            pooled_v.at[g % 2],
            out_hbm.at[pl.ds(out_base, NBUF * ROWS_PER_CHUNK)], osem)
        return carry

    lax.fori_loop(0, CHUNKS // NBUF, step_body, 0)
    for _ in range(2):
        pltpu.make_async_copy(
            pooled_v.at[0], out_hbm.at[pl.ds(0, NBUF * ROWS_PER_CHUNK)],
            osem).wait()


def _mlp_body(x_ref, w1_ref, b1_ref, w2_ref, b2_ref, w3_ref, b3_ref, o_ref):
    x = x_ref[...]
    h = jnp.maximum(
        jnp.dot(x, w1_ref[...], preferred_element_type=jnp.float32)
        + b1_ref[...], 0.0)
    h = jnp.maximum(
        jnp.dot(h, w2_ref[...], preferred_element_type=jnp.float32)
        + b2_ref[...], 0.0)
    o_ref[...] = (
        jnp.dot(h, w3_ref[...], preferred_element_type=jnp.float32)
        + b3_ref[...])


def kernel(node_sequence, table, W1, b1, W2, b2, W3, b3):
    idx = node_sequence.astype(jnp.int32).reshape(NW, CHUNKS, IDX_PER_CHUNK)

    mesh = plsc.VectorSubcoreMesh(core_axis_name="c", subcore_axis_name="s")
    pooled = pl.kernel(
        _sc_pool,
        mesh=mesh,
        out_type=jax.ShapeDtypeStruct((B, D), jnp.float32),
        scratch_types=[
            pltpu.VMEM((CHUNKS, IDX_PER_CHUNK), jnp.int32),
            pltpu.VMEM((NBUF, IDX_PER_CHUNK, D), jnp.float32),
            pltpu.VMEM((2, NBUF * ROWS_PER_CHUNK, D), jnp.float32),
            pltpu.SemaphoreType.DMA,
            pltpu.SemaphoreType.DMA,
            pltpu.SemaphoreType.DMA,
        ],
    )(idx, table)

    bt = 4096
    out = pl.pallas_call(
        _mlp_body,
        grid=(B // bt,),
        in_specs=[
            pl.BlockSpec((bt, D), lambda i: (i, 0)),
            pl.BlockSpec((D, 128), lambda i: (0, 0)),
            pl.BlockSpec((1, 128), lambda i: (0, 0)),
            pl.BlockSpec((128, 64), lambda i: (0, 0)),
            pl.BlockSpec((1, 64), lambda i: (0, 0)),
            pl.BlockSpec((64, 1), lambda i: (0, 0)),
            pl.BlockSpec((1, 1), lambda i: (0, 0)),
        ],
        out_specs=pl.BlockSpec((bt, 1), lambda i: (i, 0)),
        out_shape=jax.ShapeDtypeStruct((B, 1), jnp.float32),
    )(pooled, W1, b1.reshape(1, 128), W2, b2.reshape(1, 64),
      W3, b3.reshape(1, 1))
    return out


# speedup vs baseline: 1.3368x; 1.2620x over previous
import functools

import jax
import jax.numpy as jnp
from jax import lax
from jax.experimental import pallas as pl
from jax.experimental.pallas import tpu as pltpu
from jax.experimental.pallas import tpu_sc as plsc

B = 4096
SEQ = 50
D = 128
NL = 16
NW = 32
ROWS_PER_W = B // NW
ROWS_PER_CHUNK = 2
IDX_PER_CHUNK = ROWS_PER_CHUNK * SEQ
CHUNKS = ROWS_PER_W // ROWS_PER_CHUNK


NBUF = 4


def _sc_pool(idx_hbm, table_hbm, out_hbm, idx_v, rows_v, pooled_v,
             sem0, sem1, sem2, sem3, osem):
    c = lax.axis_index("c")
    s = lax.axis_index("s")
    wid = s * 2 + c
    sems = (sem0, sem1, sem2, sem3)
    pltpu.sync_copy(idx_hbm.at[wid], idx_v)

    for b in range(NBUF):
        pltpu.async_copy(table_hbm.at[idx_v.at[b]], rows_v.at[b], sems[b])

    def step_body(g, carry):
        @pl.when(g >= 2)
        def _():
            pltpu.make_async_copy(
                pooled_v.at[0], out_hbm.at[pl.ds(0, NBUF * ROWS_PER_CHUNK)],
                osem).wait()
        for b in range(NBUF):
            j = g * NBUF + b
            pltpu.make_async_copy(
                table_hbm.at[idx_v.at[0]], rows_v.at[b], sems[b]).wait()
            for r in range(ROWS_PER_CHUNK):
                def seq_body(t, accs, r=r, b=b):
                    base = r * SEQ + t * 10
                    out = list(accs)
                    for tt in range(10):
                        for l in range(D // NL):
                            out[l] = out[l] + rows_v[b, base + tt,
                                                     pl.ds(l * NL, NL)]
                    return tuple(out)

                accs = lax.fori_loop(
                    0, SEQ // 10, seq_body,
                    tuple(jnp.zeros((NL,), jnp.float32)
                          for _ in range(D // NL)))
                for l in range(D // NL):
                    pooled_v[g % 2, b * ROWS_PER_CHUNK + r,
                             pl.ds(l * NL, NL)] = (accs[l] * (1.0 / SEQ))
            @pl.when(j + NBUF < CHUNKS)
            def _():
                pltpu.async_copy(
                    table_hbm.at[idx_v.at[j + NBUF]], rows_v.at[b], sems[b])
        out_base = wid * ROWS_PER_W + g * (NBUF * ROWS_PER_CHUNK)
        pltpu.async_copy(
            pooled_v.at[g % 2],
            out_hbm.at[pl.ds(out_base, NBUF * ROWS_PER_CHUNK)], osem)
        return carry

    lax.fori_loop(0, CHUNKS // NBUF, step_body, 0)
    for _ in range(2):
        pltpu.make_async_copy(
            pooled_v.at[0], out_hbm.at[pl.ds(0, NBUF * ROWS_PER_CHUNK)],
            osem).wait()


def _mlp_body(x_ref, w1_ref, b1_ref, w2_ref, b2_ref, w3_ref, b3_ref, o_ref):
    x = x_ref[...]
    h = jnp.maximum(
        jnp.dot(x, w1_ref[...], preferred_element_type=jnp.float32)
        + b1_ref[...], 0.0)
    h = jnp.maximum(
        jnp.dot(h, w2_ref[...], preferred_element_type=jnp.float32)
        + b2_ref[...], 0.0)
    o_ref[...] = (
        jnp.dot(h, w3_ref[...], preferred_element_type=jnp.float32)
        + b3_ref[...])


def kernel(node_sequence, table, W1, b1, W2, b2, W3, b3):
    idx = node_sequence.astype(jnp.int32).reshape(NW, CHUNKS, IDX_PER_CHUNK)

    mesh = plsc.VectorSubcoreMesh(core_axis_name="c", subcore_axis_name="s")
    pooled = pl.kernel(
        _sc_pool,
        mesh=mesh,
        out_type=jax.ShapeDtypeStruct((B, D), jnp.float32),
        scratch_types=[
            pltpu.VMEM((CHUNKS, IDX_PER_CHUNK), jnp.int32),
            pltpu.VMEM((NBUF, IDX_PER_CHUNK, D), jnp.float32),
            pltpu.VMEM((2, NBUF * ROWS_PER_CHUNK, D), jnp.float32),
            pltpu.SemaphoreType.DMA,
            pltpu.SemaphoreType.DMA,
            pltpu.SemaphoreType.DMA,
            pltpu.SemaphoreType.DMA,
            pltpu.SemaphoreType.DMA,
        ],
    )(idx, table)

    bt = 4096
    out = pl.pallas_call(
        _mlp_body,
        grid=(B // bt,),
        in_specs=[
            pl.BlockSpec((bt, D), lambda i: (i, 0)),
            pl.BlockSpec((D, 128), lambda i: (0, 0)),
            pl.BlockSpec((1, 128), lambda i: (0, 0)),
            pl.BlockSpec((128, 64), lambda i: (0, 0)),
            pl.BlockSpec((1, 64), lambda i: (0, 0)),
            pl.BlockSpec((64, 1), lambda i: (0, 0)),
            pl.BlockSpec((1, 1), lambda i: (0, 0)),
        ],
        out_specs=pl.BlockSpec((bt, 1), lambda i: (i, 0)),
        out_shape=jax.ShapeDtypeStruct((B, 1), jnp.float32),
    )(pooled, W1, b1.reshape(1, 128), W2, b2.reshape(1, 64),
      W3, b3.reshape(1, 1))
    return out
